# baseline (speedup 1.0000x reference)
import functools

import jax
import jax.numpy as jnp
from jax import lax
from jax.experimental import pallas as pl
from jax.experimental.pallas import tpu as pltpu
from jax.experimental.pallas import tpu_sc as plsc


def _sc_gather_fn(n_nodes, dim_node, per_w, chunk, n_chunks, nc):
    mesh = plsc.VectorSubcoreMesh(core_axis_name="c", subcore_axis_name="s", num_cores=1)

    @functools.partial(
        pl.kernel,
        mesh=mesh,
        out_type=jax.ShapeDtypeStruct((n_nodes, dim_node), jnp.float32),
        scratch_types=[
            pltpu.VMEM((per_w,), jnp.int32),
            pltpu.VMEM((per_w, dim_node), jnp.float32),
            pltpu.VMEM_SHARED((89, dim_node), jnp.float32),
            pltpu.SemaphoreType.DMA,
        ],
    )
    def sc_gather(node_idx_hbm, node_table_hbm, node_out_hbm,
                  idx_v, rows_v, tab_sh, sem):
        wid = lax.axis_index("s") * nc + lax.axis_index("c")
        base = jnp.minimum(wid * per_w, n_nodes - per_w)
        pltpu.sync_copy(node_idx_hbm.at[pl.ds(base, per_w)], idx_v)

        @pl.when(wid == 0)
        def _():
            pltpu.sync_copy(node_table_hbm, tab_sh)

        plsc.subcore_barrier()
        copies = []
        for j in range(n_chunks):
            copies.append(
                pltpu.async_copy(
                    tab_sh.at[idx_v.at[pl.ds(j * chunk, chunk)]],
                    rows_v.at[pl.ds(j * chunk, chunk)],
                    sem,
                )
            )
        for cp in copies:
            cp.wait()
        pltpu.sync_copy(rows_v, node_out_hbm.at[pl.ds(base, per_w)])

    return sc_gather


def _edge_mlp_body(sidx_ref, at_ref, w_ref, b_ref, st_ref, o_ref, so_ref):
    acc = lax.dot_general(
        at_ref[...], w_ref[...],
        dimension_numbers=(((0,), (0,)), ((), ())),
        preferred_element_type=jnp.float32,
    )
    o_ref[...] = jnp.maximum(acc + b_ref[...], 0.0)

    @pl.when(pl.program_id(0) == 0)
    def _():
        so_ref[...] = st_ref[pl.ds(sidx_ref[0], 1), :]


def kernel(node_attr, edge_attr, state_attr, node_table, edge_W, edge_b, state_table):
    n_nodes = node_attr.shape[0]
    dim_node = node_table.shape[1]
    n_edges, deg = edge_attr.shape
    dim_edge = edge_W.shape[1]
    n_state, dim_state = state_table.shape

    info = plsc.get_sparse_core_info()
    nw = 1 * info.num_subcores
    quantum = 128 * nw
    per_w = ((n_nodes + quantum - 1) // quantum) * quantum // nw
    chunk = 128
    n_chunks = per_w // chunk

    sc_gather = _sc_gather_fn(n_nodes, dim_node, per_w, chunk, n_chunks, 1)
    node_feat = sc_gather(node_attr.astype(jnp.int32), node_table)

    blk = 16000
    while n_edges % blk or blk % 128:
        blk //= 2
    grid = n_edges // blk
    edge_feat, state_feat = pl.pallas_call(
        _edge_mlp_body,
        grid=(grid,),
        in_specs=[
            pl.BlockSpec(memory_space=pltpu.SMEM),
            pl.BlockSpec((deg, blk), lambda i: (0, i)),
            pl.BlockSpec((deg, dim_edge), lambda i: (0, 0)),
            pl.BlockSpec((1, dim_edge), lambda i: (0, 0)),
            pl.BlockSpec((n_state, dim_state), lambda i: (0, 0)),
        ],
        out_specs=[
            pl.BlockSpec((blk, dim_edge), lambda i: (i, 0)),
            pl.BlockSpec((1, dim_state), lambda i: (0, 0)),
        ],
        out_shape=[
            jax.ShapeDtypeStruct((n_edges, dim_edge), jnp.float32),
            jax.ShapeDtypeStruct((1, dim_state), jnp.float32),
        ],
    )(state_attr.astype(jnp.int32), edge_attr.astype(jnp.float32).T,
      edge_W, edge_b.reshape(1, dim_edge), state_table)

    return (node_feat, edge_feat, state_feat)


# speedup vs baseline: 1.0247x; 1.0247x over previous
import functools

import jax
import jax.numpy as jnp
from jax import lax
from jax.experimental import pallas as pl
from jax.experimental.pallas import tpu as pltpu
from jax.experimental.pallas import tpu_sc as plsc


def _sc_gather_fn(n_nodes, n_rows_tab, dim_node, per_w, chunk, n_chunks, nc):
    mesh = plsc.VectorSubcoreMesh(core_axis_name="c", subcore_axis_name="s", num_cores=1)

    @functools.partial(
        pl.kernel,
        mesh=mesh,
        out_type=jax.ShapeDtypeStruct((n_nodes, dim_node), jnp.float32),
        scratch_types=[
            pltpu.VMEM((per_w,), jnp.int32),
            pltpu.VMEM((per_w, dim_node), jnp.float32),
            pltpu.VMEM_SHARED((n_rows_tab, dim_node), jnp.float32),
            pltpu.SemaphoreType.DMA,
        ],
    )
    def sc_gather(node_idx_hbm, node_table_hbm, node_out_hbm,
                  idx_v, rows_v, tab_sh, sem):
        wid = lax.axis_index("s") * nc + lax.axis_index("c")
        base = jnp.minimum(wid * per_w, n_nodes - per_w)
        pltpu.sync_copy(node_idx_hbm.at[pl.ds(base, per_w)], idx_v)

        @pl.when(wid == 0)
        def _():
            pltpu.sync_copy(node_table_hbm, tab_sh)

        plsc.subcore_barrier()
        copies = []
        for j in range(n_chunks):
            copies.append(
                pltpu.async_copy(
                    tab_sh.at[idx_v.at[pl.ds(j * chunk, chunk)]],
                    rows_v.at[pl.ds(j * chunk, chunk)],
                    sem,
                )
            )
        for cp in copies:
            cp.wait()
        pltpu.sync_copy(rows_v, node_out_hbm.at[pl.ds(base, per_w)])

    return sc_gather


def _edge_mlp_body(sidx_ref, at_ref, w_ref, b_ref, st_ref, o_ref, so_ref):
    acc = lax.dot_general(
        at_ref[...], w_ref[...],
        dimension_numbers=(((0,), (0,)), ((), ())),
        preferred_element_type=jnp.float32,
    )
    o_ref[...] = jnp.maximum(acc + b_ref[...], 0.0)

    @pl.when(pl.program_id(0) == 0)
    def _():
        so_ref[...] = st_ref[pl.ds(sidx_ref[0], 1), :]


def kernel(node_attr, edge_attr, state_attr, node_table, edge_W, edge_b, state_table):
    n_nodes = node_attr.shape[0]
    dim_node = node_table.shape[1]
    n_edges, deg = edge_attr.shape
    dim_edge = edge_W.shape[1]
    n_state, dim_state = state_table.shape

    info = plsc.get_sparse_core_info()
    nw = 1 * info.num_subcores
    quantum = 128 * nw
    per_w = ((n_nodes + quantum - 1) // quantum) * quantum // nw
    chunk = 128
    n_chunks = per_w // chunk

    sc_gather = _sc_gather_fn(n_nodes, node_table.shape[0], dim_node,
                              per_w, chunk, n_chunks, 1)
    node_feat = sc_gather(node_attr.astype(jnp.int32), node_table)

    blk = 32000
    while n_edges % blk or blk % 128:
        blk //= 2
    grid = n_edges // blk
    edge_feat, state_feat = pl.pallas_call(
        _edge_mlp_body,
        grid=(grid,),
        in_specs=[
            pl.BlockSpec(memory_space=pltpu.SMEM),
            pl.BlockSpec((deg, blk), lambda i: (0, i)),
            pl.BlockSpec((deg, dim_edge), lambda i: (0, 0)),
            pl.BlockSpec((1, dim_edge), lambda i: (0, 0)),
            pl.BlockSpec((n_state, dim_state), lambda i: (0, 0)),
        ],
        out_specs=[
            pl.BlockSpec((blk, dim_edge), lambda i: (i, 0)),
            pl.BlockSpec((1, dim_state), lambda i: (0, 0)),
        ],
        out_shape=[
            jax.ShapeDtypeStruct((n_edges, dim_edge), jnp.float32),
            jax.ShapeDtypeStruct((1, dim_state), jnp.float32),
        ],
    )(state_attr.astype(jnp.int32), edge_attr.astype(jnp.float32).T,
      edge_W, edge_b.reshape(1, dim_edge), state_table)

    return (node_feat, edge_feat, state_feat)
